# baseline (speedup 1.0000x reference)
import jax
import jax.numpy as jnp
from jax.experimental import pallas as pl
from jax.experimental.pallas import tpu as pltpu

B, T, D, H = 8, 2048, 512, 512
S = T
TB = 256
NT = T // TB


def _main_body(x_ref, u_ref, m_ref, W1_ref, b1_ref, W2_ref, b2_ref,
               pooled_ref, peritem_ref, counts_ref, carry_ref):
    t = pl.program_id(1)

    @pl.when(t == 0)
    def _init():
        carry_ref[0, 0] = 0.0
        counts_ref[...] = jnp.zeros_like(counts_ref)
        pooled_ref[...] = jnp.zeros_like(pooled_ref)

    x = x_ref[0]
    h = jnp.maximum(
        jnp.dot(x, W1_ref[...], preferred_element_type=jnp.float32)
        + b1_ref[...], 0.0)
    logits = (jnp.dot(h, W2_ref[...], preferred_element_type=jnp.float32)
              + b2_ref[0, 0])
    u = u_ref[0, 0]
    noise = jnp.log(u) - jnp.log1p(-u)
    soft = jax.nn.sigmoid(logits + noise)
    hard = (soft > 0.5).astype(jnp.float32) * m_ref[0, 0]

    ri = jax.lax.broadcasted_iota(jnp.int32, (TB, TB), 0)
    ci = jax.lax.broadcasted_iota(jnp.int32, (TB, TB), 1)
    L = (ci <= ri).astype(jnp.bfloat16)
    cs = jnp.dot(L, hard.astype(jnp.bfloat16),
                 preferred_element_type=jnp.float32)
    carry = carry_ref[0, 0]
    seg = carry + cs - hard
    carry_ref[0, 0] = carry + cs[TB - 1, 0]

    s_iota = jax.lax.broadcasted_iota(jnp.int32, (TB, S), 1)
    A = (seg.astype(jnp.int32) == s_iota).astype(jnp.bfloat16)
    contrib = jax.lax.dot_general(
        A, x.astype(jnp.bfloat16), (((0,), (0,)), ((), ())),
        preferred_element_type=jnp.float32)
    pooled_ref[0] = pooled_ref[0] + contrib
    ones_col = jnp.ones((TB, 1), jnp.bfloat16)
    ccontrib = jax.lax.dot_general(
        A, ones_col, (((0,), (0,)), ((), ())),
        preferred_element_type=jnp.float32)
    counts_ref[...] = counts_ref[...] + ccontrib

    @pl.when(t == NT - 1)
    def _finish():
        inv = 1.0 / (counts_ref[...] + 1e-9)
        pooled_ref[0] = pooled_ref[0] * inv
        peritem_ref[...] = jnp.full((1, 1, 128), carry_ref[0, 0],
                                    dtype=jnp.float32)


def _lgamma(x):
    z = x + 8.0
    zi = 1.0 / z
    zi2 = zi * zi
    series = ((z - 0.5) * jnp.log(z) - z + 0.9189385332046727
              + zi * (1.0 / 12.0 + zi2 * (-1.0 / 360.0 + zi2 / 1260.0)))
    prod = (x * (x + 1.0) * (x + 2.0) * (x + 3.0)
            * (x + 4.0) * (x + 5.0) * (x + 6.0) * (x + 7.0))
    return series - jnp.log(prod)


def _loss_body(nb_ref, tgt_ref, m_ref,
               loss_ref, numb_ref, totpos_ref, short_ref):
    nb = nb_ref[...]
    tgt = tgt_ref[...]
    m = m_ref[...]
    totals = jnp.sum(m, axis=1, keepdims=True)
    p = jnp.clip(tgt / totals, 1e-6, 1.0 - 1e-6)
    log_prob = (_lgamma(totals + 1.0) - _lgamma(nb + 1.0)
                - _lgamma(totals - nb + 1.0)
                + nb * jnp.log(p) + (totals - nb) * jnp.log1p(-p))
    loss = -log_prob / totals
    loss_ref[...] = jnp.full((1, 1), jnp.sum(loss) / B, dtype=jnp.float32)
    numb_ref[...] = jnp.full((1, 1), jnp.sum(nb), dtype=jnp.float32)
    totpos_ref[...] = jnp.full((1, 1), jnp.sum(m), dtype=jnp.float32)
    s_iota = jax.lax.broadcasted_iota(jnp.int32, (B, S), 1).astype(jnp.float32)
    short_ref[...] = (s_iota < nb).astype(jnp.float32)


def kernel(hidden, attention_mask, target_boundary_counts,
           W1, b1, W2, b2, gumbel_u):
    gu = gumbel_u.reshape(B, NT, TB, 1)
    mk = attention_mask.reshape(B, NT, TB, 1)
    b1r = b1.reshape(1, H)
    b2r = jnp.broadcast_to(b2.reshape(1, 1), (1, 128))

    pooled, peritem = pl.pallas_call(
        _main_body,
        grid=(B, NT),
        in_specs=[
            pl.BlockSpec((1, TB, D), lambda b, t: (b, t, 0)),
            pl.BlockSpec((1, 1, TB, 1), lambda b, t: (b, t, 0, 0)),
            pl.BlockSpec((1, 1, TB, 1), lambda b, t: (b, t, 0, 0)),
            pl.BlockSpec((D, H), lambda b, t: (0, 0)),
            pl.BlockSpec((1, H), lambda b, t: (0, 0)),
            pl.BlockSpec((H, 1), lambda b, t: (0, 0)),
            pl.BlockSpec((1, 128), lambda b, t: (0, 0)),
        ],
        out_specs=[
            pl.BlockSpec((1, S, D), lambda b, t: (b, 0, 0)),
            pl.BlockSpec((1, 1, 128), lambda b, t: (b, 0, 0)),
        ],
        out_shape=[
            jax.ShapeDtypeStruct((B, S, D), jnp.float32),
            jax.ShapeDtypeStruct((B, 1, 128), jnp.float32),
        ],
        scratch_shapes=[
            pltpu.VMEM((S, 1), jnp.float32),
            pltpu.SMEM((1, 1), jnp.float32),
        ],
        compiler_params=pltpu.CompilerParams(
            dimension_semantics=("arbitrary", "arbitrary")),
    )(hidden, gu, mk, W1, b1r, W2, b2r)

    nb_col = peritem[:, 0, :1]
    tgt_col = target_boundary_counts.astype(jnp.float32).reshape(B, 1)

    loss2, numb2, totpos2, shortened = pl.pallas_call(
        _loss_body,
        out_shape=[
            jax.ShapeDtypeStruct((1, 1), jnp.float32),
            jax.ShapeDtypeStruct((1, 1), jnp.float32),
            jax.ShapeDtypeStruct((1, 1), jnp.float32),
            jax.ShapeDtypeStruct((B, S), jnp.float32),
        ],
    )(nb_col, tgt_col, attention_mask)

    return (pooled, loss2[0, 0], numb2[0, 0], totpos2[0, 0], shortened)


# speedup vs baseline: 1.2308x; 1.2308x over previous
import jax
import jax.numpy as jnp
from jax.experimental import pallas as pl
from jax.experimental.pallas import tpu as pltpu

B, T, D, H = 8, 2048, 512, 512
S = T
TB = 256
NT = T // TB


def _main_body(x_ref, u_ref, m_ref, W1_ref, b1_ref, W2_ref, b2_ref,
               pooled_ref, peritem_ref, counts_ref, carry_ref):
    t = pl.program_id(1)

    @pl.when(t == 0)
    def _init():
        carry_ref[0, 0] = 0.0
        counts_ref[...] = jnp.zeros_like(counts_ref)
        pooled_ref[...] = jnp.zeros_like(pooled_ref)

    x = x_ref[0]
    h = jnp.maximum(
        jnp.dot(x, W1_ref[...], preferred_element_type=jnp.float32)
        + b1_ref[...], 0.0)
    logits = (jnp.dot(h, W2_ref[...], preferred_element_type=jnp.float32)
              + b2_ref[0, 0])
    u = u_ref[0, 0]
    noise = jnp.log(u) - jnp.log1p(-u)
    soft = jax.nn.sigmoid(logits + noise)
    hard = (soft > 0.5).astype(jnp.float32) * m_ref[0, 0]

    ri = jax.lax.broadcasted_iota(jnp.int32, (TB, TB), 0)
    ci = jax.lax.broadcasted_iota(jnp.int32, (TB, TB), 1)
    L = (ci <= ri).astype(jnp.bfloat16)
    cs = jnp.dot(L, hard.astype(jnp.bfloat16),
                 preferred_element_type=jnp.float32)
    carry = carry_ref[0, 0]
    seg = carry + cs - hard
    carry_ref[0, 0] = carry + cs[TB - 1, 0]

    W = 2 * TB
    carry_i = carry.astype(jnp.int32)
    base = jnp.minimum((carry_i // TB) * TB, S - W)
    base = pl.multiple_of(base, TB)
    rel = seg.astype(jnp.int32) - base
    s_iota = jax.lax.broadcasted_iota(jnp.int32, (TB, W), 1)
    A = (rel == s_iota).astype(jnp.bfloat16)
    contrib = jax.lax.dot_general(
        A, x.astype(jnp.bfloat16), (((0,), (0,)), ((), ())),
        preferred_element_type=jnp.float32)
    pooled_ref[0, pl.ds(base, W), :] = pooled_ref[0, pl.ds(base, W), :] + contrib
    ones_col = jnp.ones((TB, 1), jnp.bfloat16)
    ccontrib = jax.lax.dot_general(
        A, ones_col, (((0,), (0,)), ((), ())),
        preferred_element_type=jnp.float32)
    counts_ref[pl.ds(base, W), :] = counts_ref[pl.ds(base, W), :] + ccontrib

    @pl.when(t == NT - 1)
    def _finish():
        inv = 1.0 / (counts_ref[...] + 1e-9)
        pooled_ref[0] = pooled_ref[0] * inv
        peritem_ref[...] = jnp.full((1, 1, 128), carry_ref[0, 0],
                                    dtype=jnp.float32)


def _lgamma(x):
    z = x + 8.0
    zi = 1.0 / z
    zi2 = zi * zi
    series = ((z - 0.5) * jnp.log(z) - z + 0.9189385332046727
              + zi * (1.0 / 12.0 + zi2 * (-1.0 / 360.0 + zi2 / 1260.0)))
    prod = (x * (x + 1.0) * (x + 2.0) * (x + 3.0)
            * (x + 4.0) * (x + 5.0) * (x + 6.0) * (x + 7.0))
    return series - jnp.log(prod)


def _loss_body(nb_ref, tgt_ref, m_ref,
               loss_ref, numb_ref, totpos_ref, short_ref):
    nb = nb_ref[...]
    tgt = tgt_ref[...]
    m = m_ref[...]
    totals = jnp.sum(m, axis=1, keepdims=True)
    p = jnp.clip(tgt / totals, 1e-6, 1.0 - 1e-6)
    log_prob = (_lgamma(totals + 1.0) - _lgamma(nb + 1.0)
                - _lgamma(totals - nb + 1.0)
                + nb * jnp.log(p) + (totals - nb) * jnp.log1p(-p))
    loss = -log_prob / totals
    loss_ref[...] = jnp.full((1, 1), jnp.sum(loss) / B, dtype=jnp.float32)
    numb_ref[...] = jnp.full((1, 1), jnp.sum(nb), dtype=jnp.float32)
    totpos_ref[...] = jnp.full((1, 1), jnp.sum(m), dtype=jnp.float32)
    s_iota = jax.lax.broadcasted_iota(jnp.int32, (B, S), 1).astype(jnp.float32)
    short_ref[...] = (s_iota < nb).astype(jnp.float32)


def kernel(hidden, attention_mask, target_boundary_counts,
           W1, b1, W2, b2, gumbel_u):
    gu = gumbel_u.reshape(B, NT, TB, 1)
    mk = attention_mask.reshape(B, NT, TB, 1)
    b1r = b1.reshape(1, H)
    b2r = jnp.broadcast_to(b2.reshape(1, 1), (1, 128))

    pooled, peritem = pl.pallas_call(
        _main_body,
        grid=(B, NT),
        in_specs=[
            pl.BlockSpec((1, TB, D), lambda b, t: (b, t, 0)),
            pl.BlockSpec((1, 1, TB, 1), lambda b, t: (b, t, 0, 0)),
            pl.BlockSpec((1, 1, TB, 1), lambda b, t: (b, t, 0, 0)),
            pl.BlockSpec((D, H), lambda b, t: (0, 0)),
            pl.BlockSpec((1, H), lambda b, t: (0, 0)),
            pl.BlockSpec((H, 1), lambda b, t: (0, 0)),
            pl.BlockSpec((1, 128), lambda b, t: (0, 0)),
        ],
        out_specs=[
            pl.BlockSpec((1, S, D), lambda b, t: (b, 0, 0)),
            pl.BlockSpec((1, 1, 128), lambda b, t: (b, 0, 0)),
        ],
        out_shape=[
            jax.ShapeDtypeStruct((B, S, D), jnp.float32),
            jax.ShapeDtypeStruct((B, 1, 128), jnp.float32),
        ],
        scratch_shapes=[
            pltpu.VMEM((S, 1), jnp.float32),
            pltpu.SMEM((1, 1), jnp.float32),
        ],
        compiler_params=pltpu.CompilerParams(
            dimension_semantics=("arbitrary", "arbitrary")),
    )(hidden, gu, mk, W1, b1r, W2, b2r)

    nb_col = peritem[:, 0, :1]
    tgt_col = target_boundary_counts.astype(jnp.float32).reshape(B, 1)

    loss2, numb2, totpos2, shortened = pl.pallas_call(
        _loss_body,
        out_shape=[
            jax.ShapeDtypeStruct((1, 1), jnp.float32),
            jax.ShapeDtypeStruct((1, 1), jnp.float32),
            jax.ShapeDtypeStruct((1, 1), jnp.float32),
            jax.ShapeDtypeStruct((B, S), jnp.float32),
        ],
    )(nb_col, tgt_col, attention_mask)

    return (pooled, loss2[0, 0], numb2[0, 0], totpos2[0, 0], shortened)
